# baseline (speedup 1.0000x reference)
import functools

import jax
import jax.numpy as jnp
from jax import lax
from jax.experimental import pallas as pl
from jax.experimental.pallas import tpu as pltpu
from jax.experimental.pallas import tpu_sc as plsc

VOCAB = 1000000
BATCH = 16384
SEQ = 50
DIM = 64
NV = DIM // 16
G = 8
NBUF = 2

NSLAB_FULL = VOCAB // 128
TAIL_COL = NSLAB_FULL * 128


def _transpose_body(tab_t_hbm, out_hbm, vbuf0, vbuf1, obuf0, obuf1, tbuf,
                    sem0, sem1):
    info = plsc.get_sparse_core_info()
    nc = info.num_cores
    nw = nc * info.num_subcores
    wid = lax.axis_index("s") * nc + lax.axis_index("c")
    lane = lax.iota(jnp.int32, 16)
    sems = (sem0, sem1)
    vbufs = (vbuf0, vbuf1)
    obufs = (obuf0, obuf1)
    niter = NSLAB_FULL // nw + 1

    def slab_copy(slab, b):
        col0 = pl.multiple_of(slab * 128, 128)
        return pltpu.make_async_copy(
            tab_t_hbm.at[pl.ds(0, DIM), pl.ds(col0, 128)], vbufs[b], sems[b])

    def transpose_block(src, b, i0, i1):
        def ti(i, _):
            sp = jnp.full((16,), i, jnp.int32)
            ob = jnp.full((16,), (i - i0) * DIM, jnp.int32) + lane
            for k in range(NV):
                v = plsc.load_gather(src, [lane + 16 * k, sp])
                plsc.store_scatter(obufs[b], [ob + 16 * k], v)
            return 0
        lax.fori_loop(i0, i1, ti, 0, unroll=2)

    for b in range(NBUF):
        slab = wid + nw * b

        @pl.when(slab < NSLAB_FULL)
        def _():
            slab_copy(slab, b).start()

    def outer(i2, _):
        for b in range(NBUF):
            i = i2 * NBUF + b
            slab = wid + nw * i

            @pl.when(slab < NSLAB_FULL)
            def _():
                slab_copy(slab, b).wait()
                transpose_block(vbufs[b], b, 0, 128)

                nxt = slab + nw * NBUF

                @pl.when(nxt < NSLAB_FULL)
                def _():
                    slab_copy(nxt, b).start()

                pltpu.sync_copy(
                    obufs[b],
                    out_hbm.at[pl.ds(pl.multiple_of(slab * 128 * DIM, 8192),
                                     128 * DIM)])
        return 0

    lax.fori_loop(0, niter // NBUF + 1, outer, 0)

    @pl.when(wid == 0)
    def _():
        c = pltpu.make_async_copy(
            tab_t_hbm.at[pl.ds(0, DIM), pl.ds(TAIL_COL, 64)], tbuf, sems[0])
        c.start()
        c.wait()
        transpose_block(tbuf, 0, 0, 64)
        pltpu.sync_copy(
            obuf0.at[pl.ds(0, 64 * DIM)],
            out_hbm.at[pl.ds(TAIL_COL * DIM, 64 * DIM)])


def _lookup_body(prem_hbm, hyp_hbm, table_hbm, wvec_hbm, out_hbm,
                 idxp_v, idxh_v, rows_v, w_v, dot_v, out_v, sem0, sem1):
    info = plsc.get_sparse_core_info()
    nc = info.num_cores
    wid = lax.axis_index("s") * nc + lax.axis_index("c")
    bpw = BATCH // (nc * info.num_subcores)
    nchunk = bpw // G
    base = pl.multiple_of(wid * bpw, bpw)
    sems = (sem0, sem1)

    pltpu.sync_copy(wvec_hbm, w_v)
    bias = w_v[pl.ds(2 * DIM, 16)][0]
    lane = lax.iota(jnp.int32, 16)
    lane_lo = lane & 7
    store_mask = lane < 8
    col_base = lane_lo * 16

    def gather_copies(g, b):
        copies = []
        for j in range(G):
            roff = (b * G + j) * 2 * SEQ
            copies.append(pltpu.make_async_copy(
                table_hbm.at[idxp_v.at[b, j]],
                rows_v.at[pl.ds(roff, SEQ)], sems[b]))
            copies.append(pltpu.make_async_copy(
                table_hbm.at[idxh_v.at[b, j]],
                rows_v.at[pl.ds(roff + SEQ, SEQ)], sems[b]))
        return copies

    def load_indices(g, b):
        row0 = base + g * G
        pltpu.sync_copy(prem_hbm.at[pl.ds(row0, G)], idxp_v.at[b])
        pltpu.sync_copy(hyp_hbm.at[pl.ds(row0, G)], idxh_v.at[b])

    def fire(g, b):
        for c in gather_copies(g, b):
            c.start()

    def drain(g, b):
        for c in gather_copies(g, b):
            c.wait()

    neg_inf = jnp.full((16,), -jnp.inf, jnp.float32)

    def compute(g, b):
        for j in range(G):
            roff = (b * G + j) * 2 * SEQ

            def seq_body(s, acc):
                new = tuple(
                    jnp.maximum(acc[d], rows_v[roff + s, pl.ds(d * 16, 16)])
                    for d in range(NV)
                ) + tuple(
                    jnp.maximum(acc[NV + d],
                                rows_v[roff + SEQ + s, pl.ds(d * 16, 16)])
                    for d in range(NV)
                )
                return new

            acc = lax.fori_loop(0, SEQ, seq_body, (neg_inf,) * (2 * NV),
                                unroll=5)

            sv = acc[0] * w_v[pl.ds(0, 16)]
            for d in range(1, NV):
                sv = sv + acc[d] * w_v[pl.ds(d * 16, 16)]
            for d in range(NV):
                sv = sv + acc[NV + d] * w_v[pl.ds(DIM + d * 16, 16)]
            dot_v[pl.ds(j * 16, 16)] = sv

        tot = plsc.load_gather(dot_v, [col_base])
        for l in range(1, 16):
            tot = tot + plsc.load_gather(dot_v, [col_base + l])
        tot = tot + bias
        plsc.store_scatter(out_v, [g * G + lane_lo], tot, mask=store_mask)

    for b in range(NBUF):
        load_indices(b, b)
        fire(b, b)

    def outer(i, _):
        g0 = i * NBUF
        for b in range(NBUF):
            g = g0 + b
            drain(g, b)
            compute(g, b)

            @pl.when(g + NBUF < nchunk)
            def _():
                load_indices(g + NBUF, b)
                fire(g + NBUF, b)
        return 0

    lax.fori_loop(0, nchunk // NBUF, outer, 0)

    def sig_body(k, _):
        iv = k * 16 + lane
        x = plsc.load_gather(out_v, [iv])
        plsc.store_scatter(out_v, [iv], 1.0 / (1.0 + jnp.exp(-x)))
        return 0

    lax.fori_loop(0, bpw // 16, sig_body, 0)
    pltpu.sync_copy(out_v, out_hbm.at[pl.ds(base, bpw)])


def kernel(premise, hypothesis, table, W, b):
    info = plsc.get_sparse_core_info()
    nw = info.num_cores * info.num_subcores
    bpw = BATCH // nw

    mesh = plsc.VectorSubcoreMesh(core_axis_name="c", subcore_axis_name="s")

    relayout = functools.partial(
        pl.kernel,
        out_type=jax.ShapeDtypeStruct((VOCAB * DIM,), jnp.float32),
        mesh=mesh,
        compiler_params=pltpu.CompilerParams(
            needs_layout_passes=False, use_tc_tiling_on_sc=True),
        scratch_types=[
            pltpu.VMEM((DIM, 128), jnp.float32),
            pltpu.VMEM((DIM, 128), jnp.float32),
            pltpu.VMEM((128 * DIM,), jnp.float32),
            pltpu.VMEM((128 * DIM,), jnp.float32),
            pltpu.VMEM((DIM, 64), jnp.float32),
            pltpu.SemaphoreType.DMA,
            pltpu.SemaphoreType.DMA,
        ],
    )(_transpose_body)
    table_lin = relayout(table.T).reshape(VOCAB, DIM)

    wvec = jnp.zeros((144,), jnp.float32)
    wvec = wvec.at[: 2 * DIM].set(W.reshape(-1)).at[2 * DIM].set(b[0])

    lookup = functools.partial(
        pl.kernel,
        out_type=jax.ShapeDtypeStruct((BATCH,), jnp.float32),
        mesh=mesh,
        compiler_params=pltpu.CompilerParams(
            needs_layout_passes=False, use_tc_tiling_on_sc=False),
        scratch_types=[
            pltpu.VMEM((NBUF, G, SEQ), jnp.int32),
            pltpu.VMEM((NBUF, G, SEQ), jnp.int32),
            pltpu.VMEM((NBUF * G * 2 * SEQ, DIM), jnp.float32),
            pltpu.VMEM((144,), jnp.float32),
            pltpu.VMEM((G * 16,), jnp.float32),
            pltpu.VMEM((bpw,), jnp.float32),
            pltpu.SemaphoreType.DMA,
            pltpu.SemaphoreType.DMA,
        ],
    )(_lookup_body)
    return lookup(premise.astype(jnp.int32), hypothesis.astype(jnp.int32),
                  table_lin, wvec)


# speedup vs baseline: 2.6975x; 2.6975x over previous
import functools

import jax
import jax.numpy as jnp
from jax import lax
from jax.experimental import pallas as pl
from jax.experimental.pallas import tpu as pltpu
from jax.experimental.pallas import tpu_sc as plsc

VOCAB = 1000000
BATCH = 16384
SEQ = 50
DIM = 64
NV = DIM // 16
G = 8
NBUF = 2

def _lookup_body(prem_hbm, hyp_hbm, table_hbm, wvec_hbm, out_hbm,
                 idxp_v, idxh_v, rows_v, w_v, dot_v, out_v, sem0, sem1):
    info = plsc.get_sparse_core_info()
    nc = info.num_cores
    wid = lax.axis_index("s") * nc + lax.axis_index("c")
    bpw = BATCH // (nc * info.num_subcores)
    nchunk = bpw // G
    base = pl.multiple_of(wid * bpw, bpw)
    sems = (sem0, sem1)

    pltpu.sync_copy(wvec_hbm, w_v)
    bias = w_v[pl.ds(2 * DIM, 16)][0]
    lane = lax.iota(jnp.int32, 16)
    lane_lo = lane & 7
    store_mask = lane < 8
    col_base = lane_lo * 16

    def gather_copies(g, b):
        copies = []
        for j in range(G):
            roff = (b * G + j) * 2 * SEQ
            copies.append(pltpu.make_async_copy(
                table_hbm.at[idxp_v.at[b, j]],
                rows_v.at[pl.ds(roff, SEQ)], sems[b]))
            copies.append(pltpu.make_async_copy(
                table_hbm.at[idxh_v.at[b, j]],
                rows_v.at[pl.ds(roff + SEQ, SEQ)], sems[b]))
        return copies

    def load_indices(g, b):
        row0 = base + g * G
        pltpu.sync_copy(prem_hbm.at[pl.ds(row0, G)], idxp_v.at[b])
        pltpu.sync_copy(hyp_hbm.at[pl.ds(row0, G)], idxh_v.at[b])

    def fire(g, b):
        for c in gather_copies(g, b):
            c.start()

    def drain(g, b):
        for c in gather_copies(g, b):
            c.wait()

    neg_inf = jnp.full((16,), -jnp.inf, jnp.float32)

    def compute(g, b):
        for j in range(G):
            roff = (b * G + j) * 2 * SEQ

            def seq_body(s, acc):
                new = tuple(
                    jnp.maximum(acc[d], rows_v[roff + s, pl.ds(d * 16, 16)])
                    for d in range(NV)
                ) + tuple(
                    jnp.maximum(acc[NV + d],
                                rows_v[roff + SEQ + s, pl.ds(d * 16, 16)])
                    for d in range(NV)
                )
                return new

            acc = lax.fori_loop(0, SEQ, seq_body, (neg_inf,) * (2 * NV),
                                unroll=5)

            sv = acc[0] * w_v[pl.ds(0, 16)]
            for d in range(1, NV):
                sv = sv + acc[d] * w_v[pl.ds(d * 16, 16)]
            for d in range(NV):
                sv = sv + acc[NV + d] * w_v[pl.ds(DIM + d * 16, 16)]
            dot_v[pl.ds(j * 16, 16)] = sv

        tot = plsc.load_gather(dot_v, [col_base])
        for l in range(1, 16):
            tot = tot + plsc.load_gather(dot_v, [col_base + l])
        tot = tot + bias
        plsc.store_scatter(out_v, [g * G + lane_lo], tot, mask=store_mask)

    for b in range(NBUF):
        load_indices(b, b)
        fire(b, b)

    def outer(i, _):
        g0 = i * NBUF
        for b in range(NBUF):
            g = g0 + b
            drain(g, b)
            compute(g, b)

            @pl.when(g + NBUF < nchunk)
            def _():
                load_indices(g + NBUF, b)
                fire(g + NBUF, b)
        return 0

    lax.fori_loop(0, nchunk // NBUF, outer, 0)

    def sig_body(k, _):
        iv = k * 16 + lane
        x = plsc.load_gather(out_v, [iv])
        plsc.store_scatter(out_v, [iv], 1.0 / (1.0 + jnp.exp(-x)))
        return 0

    lax.fori_loop(0, bpw // 16, sig_body, 0)
    pltpu.sync_copy(out_v, out_hbm.at[pl.ds(base, bpw)])


def kernel(premise, hypothesis, table, W, b):
    info = plsc.get_sparse_core_info()
    nw = info.num_cores * info.num_subcores
    bpw = BATCH // nw

    mesh = plsc.VectorSubcoreMesh(core_axis_name="c", subcore_axis_name="s")

    BLK = 16384
    grid = (VOCAB + BLK - 1) // BLK

    def _tc_transpose(in_ref, out_ref):
        xt = in_ref[...].T.reshape(BLK // 2, 2, DIM)
        out_ref[...] = jnp.concatenate([xt[:, 0, :], xt[:, 1, :]], axis=1)

    table_wide = pl.pallas_call(
        _tc_transpose,
        grid=(grid,),
        in_specs=[pl.BlockSpec((DIM, BLK), lambda g: (0, g))],
        out_specs=pl.BlockSpec((BLK // 2, 128), lambda g: (g, 0)),
        out_shape=jax.ShapeDtypeStruct((VOCAB // 2, 128), jnp.float32),
    )(table.T)
    table_lin = table_wide.reshape(VOCAB, DIM)

    wvec = jnp.zeros((144,), jnp.float32)
    wvec = wvec.at[: 2 * DIM].set(W.reshape(-1)).at[2 * DIM].set(b[0])

    lookup = functools.partial(
        pl.kernel,
        out_type=jax.ShapeDtypeStruct((BATCH,), jnp.float32),
        mesh=mesh,
        compiler_params=pltpu.CompilerParams(
            needs_layout_passes=False, use_tc_tiling_on_sc=False),
        scratch_types=[
            pltpu.VMEM((NBUF, G, SEQ), jnp.int32),
            pltpu.VMEM((NBUF, G, SEQ), jnp.int32),
            pltpu.VMEM((NBUF * G * 2 * SEQ, DIM), jnp.float32),
            pltpu.VMEM((144,), jnp.float32),
            pltpu.VMEM((G * 16,), jnp.float32),
            pltpu.VMEM((bpw,), jnp.float32),
            pltpu.SemaphoreType.DMA,
            pltpu.SemaphoreType.DMA,
        ],
    )(_lookup_body)
    return lookup(premise.astype(jnp.int32), hypothesis.astype(jnp.int32),
                  table_lin, wvec)
